# baseline (speedup 1.0000x reference)
import functools

import jax
import jax.numpy as jnp
from jax import lax
from jax.experimental import pallas as pl
from jax.experimental.pallas import tpu as pltpu
from jax.experimental.pallas import tpu_sc as plsc

_N = 10000
_E = 320000
_D = 128
_HID = 256
_EMB = 128
_OUT = 64

_NC = 2
_NS = 16
_NW = _NC * _NS
_CH = 128
_NCH = 80
_EPW = _NCH * _CH
_EP = _NW * _EPW
_NP = 10240
_RPT = _NP // _NS
_BB = 32
_NBB = _RPT // _BB

_MESH = plsc.VectorSubcoreMesh(core_axis_name="c", subcore_axis_name="s",
                               num_cores=_NC, num_subcores=_NS)


@functools.partial(
    pl.kernel,
    out_type=jax.ShapeDtypeStruct((_NC, _NP, _D), jnp.float32),
    mesh=_MESH,
    scratch_types=[
        pltpu.VMEM((_CH,), jnp.int32),
        pltpu.VMEM((_CH,), jnp.int32),
        pltpu.VMEM((_CH,), jnp.int32),
        pltpu.VMEM((_CH,), jnp.int32),
        pltpu.VMEM((_CH, _D), jnp.float32),
        pltpu.VMEM((_CH, _D), jnp.float32),
        pltpu.VMEM((_BB, _D), jnp.float32),
        pltpu.SemaphoreType.DMA,
        pltpu.SemaphoreType.DMA,
        pltpu.SemaphoreType.DMA,
        pltpu.SemaphoreType.DMA,
        pltpu.VMEM_SHARED((_NP, _D), jnp.float32),
    ],
)
def _seg(a_hbm, src_hbm, dst_hbm, zd_hbm,
         accp_hbm,
         src0_v, src1_v, dst0_v, dst1_v, rows0_v, rows1_v, bb_v,
         sem0, sem1, isem0, isem1, acc_sh):
    c = lax.axis_index("c")
    s = lax.axis_index("s")
    wid = s * _NC + c
    base = s * _RPT
    pltpu.sync_copy(zd_hbm.at[pl.ds(0, _BB)], bb_v)

    def zinit(i, carry):
        pltpu.sync_copy(bb_v, acc_sh.at[pl.ds(base + i * _BB, _BB)])
        return carry

    lax.fori_loop(0, _NBB, zinit, 0)
    plsc.subcore_barrier()

    srcs = (src0_v, src1_v)
    dsts = (dst0_v, dst1_v)
    rows = (rows0_v, rows1_v)
    sems = (sem0, sem1)
    isems = (isem0, isem1)
    ebase = wid * _EPW

    def stage_idx(k, slot):
        off = ebase + k * _CH
        pltpu.async_copy(src_hbm.at[pl.ds(off, _CH)], srcs[slot], isems[slot])
        pltpu.async_copy(dst_hbm.at[pl.ds(off, _CH)], dsts[slot], isems[slot])

    def wait_idx(k, slot):
        off = ebase + k * _CH
        pltpu.make_async_copy(src_hbm.at[pl.ds(off, _CH)], srcs[slot],
                              isems[slot]).wait()
        pltpu.make_async_copy(dst_hbm.at[pl.ds(off, _CH)], dsts[slot],
                              isems[slot]).wait()

    stage_idx(0, 0)
    stage_idx(1, 1)
    wait_idx(0, 0)
    pltpu.async_copy(a_hbm.at[srcs[0]], rows[0], sems[0])

    def round_body(jo, carry):
        for b in (0, 1):
            j = jo * 2 + b
            pltpu.make_async_copy(a_hbm.at[srcs[b]], rows[b], sems[b]).wait()

            @pl.when(j + 1 < _NCH)
            def _start_next():
                wait_idx(j + 1, 1 - b)
                pltpu.async_copy(a_hbm.at[srcs[1 - b]], rows[1 - b], sems[1 - b])

            pltpu.sync_copy(rows[b], acc_sh.at[dsts[b]], add=True)

            @pl.when(j + 2 < _NCH)
            def _prefetch_idx():
                stage_idx(j + 2, b)
        return carry

    lax.fori_loop(0, _NCH // 2, round_body, 0)
    plsc.subcore_barrier()

    def wback(i, carry):
        o = base + i * _BB
        pltpu.sync_copy(acc_sh.at[pl.ds(o, _BB)], bb_v)
        pltpu.sync_copy(bb_v, accp_hbm.at[c, pl.ds(o, _BB)])
        return carry

    lax.fori_loop(0, _NBB, wback, 0)


@functools.partial(
    pl.kernel,
    out_type=jax.ShapeDtypeStruct((_NC, _NP, _D), jnp.float32),
    mesh=_MESH,
    scratch_types=[
        pltpu.VMEM((_NCH, _CH), jnp.int32),
        pltpu.VMEM((_CH, _D), jnp.float32),
        pltpu.VMEM((_BB, _D), jnp.float32),
        pltpu.VMEM_SHARED((_NP, _D), jnp.float32),
    ],
)
def _cnt(dst_hbm, zd_hbm, ones_hbm,
         cntp_hbm,
         didx_v, ones_v, bb_v, cnt_sh):
    c = lax.axis_index("c")
    s = lax.axis_index("s")
    wid = s * _NC + c
    base = s * _RPT
    pltpu.sync_copy(ones_hbm, ones_v)
    pltpu.sync_copy(zd_hbm.at[pl.ds(0, _BB)], bb_v)

    def zinit(i, carry):
        pltpu.sync_copy(bb_v, cnt_sh.at[pl.ds(base + i * _BB, _BB)])
        return carry

    lax.fori_loop(0, _NBB, zinit, 0)
    plsc.subcore_barrier()
    pltpu.sync_copy(dst_hbm.at[wid], didx_v)

    def body(j, carry):
        pltpu.sync_copy(ones_v, cnt_sh.at[didx_v.at[j]], add=True)
        return carry

    lax.fori_loop(0, _NCH, body, 0)
    plsc.subcore_barrier()

    def wback(i, carry):
        o = base + i * _BB
        pltpu.sync_copy(cnt_sh.at[pl.ds(o, _BB)], bb_v)
        pltpu.sync_copy(bb_v, cntp_hbm.at[c, pl.ds(o, _BB)])
        return carry

    lax.fori_loop(0, _NBB, wback, 0)


_BR = 1000


def _tc1_body(x_ref, s1_ref, cnt_ref, wl1_ref, wr1_ref, b1_ref, wl2_ref,
              h_ref, p2_ref):
    cnt = jnp.maximum(cnt_ref[0][:, 0:1] + cnt_ref[1][:, 0:1], 1.0)
    mean = (s1_ref[0] + s1_ref[1]) / cnt
    h = jnp.dot(mean, wl1_ref[...], preferred_element_type=jnp.float32)
    h = h + jnp.dot(x_ref[...], wr1_ref[...], preferred_element_type=jnp.float32)
    h = jnp.maximum(h + b1_ref[...], 0.0)
    h_ref[...] = h
    p2_ref[...] = jnp.dot(h, wl2_ref[...], preferred_element_type=jnp.float32)


_tc1 = pl.pallas_call(
    _tc1_body,
    grid=(_N // _BR,),
    in_specs=[
        pl.BlockSpec((_BR, _D), lambda i: (i, 0)),
        pl.BlockSpec((_NC, _BR, _D), lambda i: (0, i, 0)),
        pl.BlockSpec((_NC, _BR, _D), lambda i: (0, i, 0)),
        pl.BlockSpec((_D, _HID), lambda i: (0, 0)),
        pl.BlockSpec((_D, _HID), lambda i: (0, 0)),
        pl.BlockSpec((1, _HID), lambda i: (0, 0)),
        pl.BlockSpec((_HID, _EMB), lambda i: (0, 0)),
    ],
    out_specs=[
        pl.BlockSpec((_BR, _HID), lambda i: (i, 0)),
        pl.BlockSpec((_BR, _EMB), lambda i: (i, 0)),
    ],
    out_shape=[
        jax.ShapeDtypeStruct((_N, _HID), jnp.float32),
        jax.ShapeDtypeStruct((_N, _EMB), jnp.float32),
    ],
)


def _tc2_body(h_ref, s2_ref, cnt_ref, wr2_ref, b2_ref, wo_ref, bo_ref,
              out_ref, h2_ref):
    cnt = jnp.maximum(cnt_ref[0][:, 0:1] + cnt_ref[1][:, 0:1], 1.0)
    mean = (s2_ref[0] + s2_ref[1]) / cnt
    h2 = mean + jnp.dot(h_ref[...], wr2_ref[...], preferred_element_type=jnp.float32)
    h2 = h2 + b2_ref[...]
    h2_ref[...] = h2
    out_ref[...] = jnp.dot(h2, wo_ref[...], preferred_element_type=jnp.float32) + bo_ref[...]


_tc2 = pl.pallas_call(
    _tc2_body,
    grid=(_N // _BR,),
    in_specs=[
        pl.BlockSpec((_BR, _HID), lambda i: (i, 0)),
        pl.BlockSpec((_NC, _BR, _D), lambda i: (0, i, 0)),
        pl.BlockSpec((_NC, _BR, _D), lambda i: (0, i, 0)),
        pl.BlockSpec((_HID, _EMB), lambda i: (0, 0)),
        pl.BlockSpec((1, _EMB), lambda i: (0, 0)),
        pl.BlockSpec((_EMB, _OUT), lambda i: (0, 0)),
        pl.BlockSpec((1, _OUT), lambda i: (0, 0)),
    ],
    out_specs=[
        pl.BlockSpec((_BR, _OUT), lambda i: (i, 0)),
        pl.BlockSpec((_BR, _EMB), lambda i: (i, 0)),
    ],
    out_shape=[
        jax.ShapeDtypeStruct((_N, _OUT), jnp.float32),
        jax.ShapeDtypeStruct((_N, _EMB), jnp.float32),
    ],
)


def kernel(x, edge_index, W_l1, W_r1, b1, W_l2, W_r2, b2, W_o, b_o):
    npad = _EP - _E
    src = jnp.concatenate([edge_index[0], jnp.zeros((npad,), jnp.int32)])
    dst = jnp.concatenate([edge_index[1], jnp.full((npad,), _N, jnp.int32)])
    dst3 = dst.reshape(_NW, _NCH, _CH)
    zd = jnp.zeros((_NP, _D), jnp.float32)
    ones = jnp.ones((_CH, _D), jnp.float32)
    cntp = _cnt(dst3, zd, ones)
    s1p = _seg(x, src, dst, zd)
    h, p2 = _tc1(x, s1p, cntp, W_l1, W_r1, b1.reshape(1, _HID), W_l2)
    s2p = _seg(p2, src, dst, zd)
    out, h2 = _tc2(h, s2p, cntp, W_r2, b2.reshape(1, _EMB), W_o,
                   b_o.reshape(1, _OUT))
    return (out, h2)


# speedup vs baseline: 2.4185x; 2.4185x over previous
import functools

import jax
import jax.numpy as jnp
from jax import lax
from jax.experimental import pallas as pl
from jax.experimental.pallas import tpu as pltpu
from jax.experimental.pallas import tpu_sc as plsc

_N = 10000
_E = 320000
_D = 128
_HID = 256
_EMB = 128
_OUT = 64

_NC = 2
_NS = 16
_NW = _NC * _NS
_EPW = _E // _NW
_CH = 80
_NCH = _EPW // _CH
_NP = 10240
_RPT = _NP // _NS
_BB = 32
_NBB = _RPT // _BB

_MESH = plsc.VectorSubcoreMesh(core_axis_name="c", subcore_axis_name="s",
                               num_cores=_NC, num_subcores=_NS)


@functools.partial(
    pl.kernel,
    out_type=jax.ShapeDtypeStruct((_NC, _NP, _D), jnp.float32),
    mesh=_MESH,
    scratch_types=[
        pltpu.VMEM((_CH,), jnp.int32),
        pltpu.VMEM((_CH,), jnp.int32),
        pltpu.VMEM((_CH,), jnp.int32),
        pltpu.VMEM((_CH,), jnp.int32),
        pltpu.VMEM((_CH, _D), jnp.float32),
        pltpu.VMEM((_CH, _D), jnp.float32),
        pltpu.VMEM((_BB, _D), jnp.float32),
        pltpu.SemaphoreType.DMA,
        pltpu.SemaphoreType.DMA,
        pltpu.SemaphoreType.DMA,
        pltpu.SemaphoreType.DMA,
        pltpu.VMEM_SHARED((_NP, _D), jnp.float32),
    ],
)
def _seg(a_hbm, src_hbm, dst_hbm, zd_hbm,
         accp_hbm,
         src0_v, src1_v, dst0_v, dst1_v, rows0_v, rows1_v, bb_v,
         sem0, sem1, isem0, isem1, acc_sh):
    c = lax.axis_index("c")
    s = lax.axis_index("s")
    wid = s * _NC + c
    base = s * _RPT
    pltpu.sync_copy(zd_hbm.at[pl.ds(0, _BB)], bb_v)

    def zinit(i, carry):
        pltpu.sync_copy(bb_v, acc_sh.at[pl.ds(base + i * _BB, _BB)])
        return carry

    lax.fori_loop(0, _NBB, zinit, 0)
    plsc.subcore_barrier()

    srcs = (src0_v, src1_v)
    dsts = (dst0_v, dst1_v)
    rows = (rows0_v, rows1_v)
    sems = (sem0, sem1)
    isems = (isem0, isem1)
    ebase = wid * _EPW

    def stage_idx(k, slot):
        off = ebase + k * _CH
        pltpu.async_copy(src_hbm.at[pl.ds(off, _CH)], srcs[slot], isems[slot])
        pltpu.async_copy(dst_hbm.at[pl.ds(off, _CH)], dsts[slot], isems[slot])

    def wait_idx(k, slot):
        off = ebase + k * _CH
        pltpu.make_async_copy(src_hbm.at[pl.ds(off, _CH)], srcs[slot],
                              isems[slot]).wait()
        pltpu.make_async_copy(dst_hbm.at[pl.ds(off, _CH)], dsts[slot],
                              isems[slot]).wait()

    stage_idx(0, 0)
    stage_idx(1, 1)
    wait_idx(0, 0)
    pltpu.async_copy(a_hbm.at[srcs[0]], rows[0], sems[0])

    def round_body(jo, carry):
        for b in (0, 1):
            j = jo * 2 + b
            pltpu.make_async_copy(a_hbm.at[srcs[b]], rows[b], sems[b]).wait()
            wait_idx(j + 1, 1 - b)
            pltpu.async_copy(a_hbm.at[srcs[1 - b]], rows[1 - b], sems[1 - b])
            pltpu.sync_copy(rows[b], acc_sh.at[dsts[b]], add=True)
            stage_idx(j + 2, b)
        return carry

    lax.fori_loop(0, (_NCH - 1) // 2, round_body, 0)
    pltpu.make_async_copy(a_hbm.at[srcs[0]], rows[0], sems[0]).wait()
    pltpu.sync_copy(rows[0], acc_sh.at[dsts[0]], add=True)
    wait_idx(_NCH, 1)
    plsc.subcore_barrier()

    def wback(i, carry):
        o = base + i * _BB
        pltpu.sync_copy(acc_sh.at[pl.ds(o, _BB)], bb_v)
        pltpu.sync_copy(bb_v, accp_hbm.at[c, pl.ds(o, _BB)])
        return carry

    lax.fori_loop(0, _NBB, wback, 0)


@functools.partial(
    pl.kernel,
    out_type=jax.ShapeDtypeStruct((_NC, _NP, _D), jnp.float32),
    mesh=_MESH,
    scratch_types=[
        pltpu.VMEM((_NCH, _CH), jnp.int32),
        pltpu.VMEM((_CH, _D), jnp.float32),
        pltpu.VMEM((_BB, _D), jnp.float32),
        pltpu.VMEM_SHARED((_NP, _D), jnp.float32),
    ],
)
def _cnt(dst_hbm, zd_hbm, ones_hbm,
         cntp_hbm,
         didx_v, ones_v, bb_v, cnt_sh):
    c = lax.axis_index("c")
    s = lax.axis_index("s")
    wid = s * _NC + c
    base = s * _RPT
    pltpu.sync_copy(ones_hbm, ones_v)
    pltpu.sync_copy(zd_hbm.at[pl.ds(0, _BB)], bb_v)

    def zinit(i, carry):
        pltpu.sync_copy(bb_v, cnt_sh.at[pl.ds(base + i * _BB, _BB)])
        return carry

    lax.fori_loop(0, _NBB, zinit, 0)
    plsc.subcore_barrier()
    pltpu.sync_copy(dst_hbm.at[wid], didx_v)

    def body(j, carry):
        pltpu.sync_copy(ones_v, cnt_sh.at[didx_v.at[j]], add=True)
        return carry

    lax.fori_loop(0, _NCH, body, 0)
    plsc.subcore_barrier()

    def wback(i, carry):
        o = base + i * _BB
        pltpu.sync_copy(cnt_sh.at[pl.ds(o, _BB)], bb_v)
        pltpu.sync_copy(bb_v, cntp_hbm.at[c, pl.ds(o, _BB)])
        return carry

    lax.fori_loop(0, _NBB, wback, 0)


_BR = 1000


def _tc1_body(x_ref, s1_ref, cnt_ref, wl1_ref, wr1_ref, b1_ref, wl2_ref,
              h_ref, p2_ref):
    cnt = jnp.maximum(cnt_ref[0][:, 0:1] + cnt_ref[1][:, 0:1], 1.0)
    mean = (s1_ref[0] + s1_ref[1]) / cnt
    h = jnp.dot(mean, wl1_ref[...], preferred_element_type=jnp.float32)
    h = h + jnp.dot(x_ref[...], wr1_ref[...], preferred_element_type=jnp.float32)
    h = jnp.maximum(h + b1_ref[...], 0.0)
    h_ref[...] = h
    p2_ref[...] = jnp.dot(h, wl2_ref[...], preferred_element_type=jnp.float32)


_tc1 = pl.pallas_call(
    _tc1_body,
    grid=(_N // _BR,),
    in_specs=[
        pl.BlockSpec((_BR, _D), lambda i: (i, 0)),
        pl.BlockSpec((_NC, _BR, _D), lambda i: (0, i, 0)),
        pl.BlockSpec((_NC, _BR, _D), lambda i: (0, i, 0)),
        pl.BlockSpec((_D, _HID), lambda i: (0, 0)),
        pl.BlockSpec((_D, _HID), lambda i: (0, 0)),
        pl.BlockSpec((1, _HID), lambda i: (0, 0)),
        pl.BlockSpec((_HID, _EMB), lambda i: (0, 0)),
    ],
    out_specs=[
        pl.BlockSpec((_BR, _HID), lambda i: (i, 0)),
        pl.BlockSpec((_BR, _EMB), lambda i: (i, 0)),
    ],
    out_shape=[
        jax.ShapeDtypeStruct((_N, _HID), jnp.float32),
        jax.ShapeDtypeStruct((_N, _EMB), jnp.float32),
    ],
)


def _tc2_body(h_ref, s2_ref, cnt_ref, wr2_ref, b2_ref, wo_ref, bo_ref,
              out_ref, h2_ref):
    cnt = jnp.maximum(cnt_ref[0][:, 0:1] + cnt_ref[1][:, 0:1], 1.0)
    mean = (s2_ref[0] + s2_ref[1]) / cnt
    h2 = mean + jnp.dot(h_ref[...], wr2_ref[...], preferred_element_type=jnp.float32)
    h2 = h2 + b2_ref[...]
    h2_ref[...] = h2
    out_ref[...] = jnp.dot(h2, wo_ref[...], preferred_element_type=jnp.float32) + bo_ref[...]


_tc2 = pl.pallas_call(
    _tc2_body,
    grid=(_N // _BR,),
    in_specs=[
        pl.BlockSpec((_BR, _HID), lambda i: (i, 0)),
        pl.BlockSpec((_NC, _BR, _D), lambda i: (0, i, 0)),
        pl.BlockSpec((_NC, _BR, _D), lambda i: (0, i, 0)),
        pl.BlockSpec((_HID, _EMB), lambda i: (0, 0)),
        pl.BlockSpec((1, _EMB), lambda i: (0, 0)),
        pl.BlockSpec((_EMB, _OUT), lambda i: (0, 0)),
        pl.BlockSpec((1, _OUT), lambda i: (0, 0)),
    ],
    out_specs=[
        pl.BlockSpec((_BR, _OUT), lambda i: (i, 0)),
        pl.BlockSpec((_BR, _EMB), lambda i: (i, 0)),
    ],
    out_shape=[
        jax.ShapeDtypeStruct((_N, _OUT), jnp.float32),
        jax.ShapeDtypeStruct((_N, _EMB), jnp.float32),
    ],
)


def kernel(x, edge_index, W_l1, W_r1, b1, W_l2, W_r2, b2, W_o, b_o):
    pad = jnp.zeros((_CH,), jnp.int32)
    src = jnp.concatenate([edge_index[0], pad])
    dst = jnp.concatenate([edge_index[1], pad])
    dst3 = edge_index[1].reshape(_NW, _NCH, _CH)
    zd = jnp.zeros((_NP, _D), jnp.float32)
    ones = jnp.ones((_CH, _D), jnp.float32)
    cntp = _cnt(dst3, zd, ones)
    s1p = _seg(x, src, dst, zd)
    h, p2 = _tc1(x, s1p, cntp, W_l1, W_r1, b1.reshape(1, _HID), W_l2)
    s2p = _seg(p2, src, dst, zd)
    out, h2 = _tc2(h, s2p, cntp, W_r2, b2.reshape(1, _EMB), W_o,
                   b_o.reshape(1, _OUT))
    return (out, h2)


# speedup vs baseline: 2.6547x; 1.0977x over previous
import functools

import jax
import jax.numpy as jnp
from jax import lax
from jax.experimental import pallas as pl
from jax.experimental.pallas import tpu as pltpu
from jax.experimental.pallas import tpu_sc as plsc

_N = 10000
_E = 320000
_D = 128
_HID = 256
_EMB = 128
_OUT = 64

_NC = 2
_NS = 16
_NW = _NC * _NS
_EPW = _E // _NW
_CH = 80
_NCH = _EPW // _CH
_NP = 10240
_RPT = _NP // _NS
_BB = 32
_NBB = _RPT // _BB

_MESH = plsc.VectorSubcoreMesh(core_axis_name="c", subcore_axis_name="s",
                               num_cores=_NC, num_subcores=_NS)


@functools.partial(
    pl.kernel,
    out_type=jax.ShapeDtypeStruct((_NC, _NP, _D), jnp.float32),
    mesh=_MESH,
    scratch_types=[
        pltpu.VMEM((_CH,), jnp.int32),
        pltpu.VMEM((_CH,), jnp.int32),
        pltpu.VMEM((_CH,), jnp.int32),
        pltpu.VMEM((_CH,), jnp.int32),
        pltpu.VMEM((_CH,), jnp.int32),
        pltpu.VMEM((_CH,), jnp.int32),
        pltpu.VMEM((_CH, _D), jnp.float32),
        pltpu.VMEM((_CH, _D), jnp.float32),
        pltpu.VMEM((_CH, _D), jnp.float32),
        pltpu.VMEM((_BB, _D), jnp.float32),
        pltpu.SemaphoreType.DMA,
        pltpu.SemaphoreType.DMA,
        pltpu.SemaphoreType.DMA,
        pltpu.SemaphoreType.DMA,
        pltpu.SemaphoreType.DMA,
        pltpu.SemaphoreType.DMA,
        pltpu.VMEM_SHARED((_NP, _D), jnp.float32),
    ],
)
def _seg(a_hbm, src_hbm, dst_hbm, zd_hbm,
         accp_hbm,
         src0_v, src1_v, src2_v, dst0_v, dst1_v, dst2_v,
         rows0_v, rows1_v, rows2_v, bb_v,
         sem0, sem1, sem2, isem0, isem1, isem2, acc_sh):
    c = lax.axis_index("c")
    s = lax.axis_index("s")
    wid = s * _NC + c
    base = s * _RPT
    pltpu.sync_copy(zd_hbm, bb_v)

    def zinit(i, carry):
        pltpu.sync_copy(bb_v, acc_sh.at[pl.ds(base + i * _BB, _BB)])
        return carry

    lax.fori_loop(0, _NBB, zinit, 0)
    plsc.subcore_barrier()

    srcs = (src0_v, src1_v, src2_v)
    dsts = (dst0_v, dst1_v, dst2_v)
    rows = (rows0_v, rows1_v, rows2_v)
    sems = (sem0, sem1, sem2)
    isems = (isem0, isem1, isem2)
    ebase = wid * _EPW

    def stage_idx(k, slot):
        kk = jnp.minimum(k, _NCH - 1)
        off = ebase + kk * _CH
        pltpu.async_copy(src_hbm.at[pl.ds(off, _CH)], srcs[slot], isems[slot])
        pltpu.async_copy(dst_hbm.at[pl.ds(off, _CH)], dsts[slot], isems[slot])

    def wait_idx(slot):
        pltpu.make_async_copy(src_hbm.at[pl.ds(0, _CH)], srcs[slot],
                              isems[slot]).wait()
        pltpu.make_async_copy(dst_hbm.at[pl.ds(0, _CH)], dsts[slot],
                              isems[slot]).wait()

    stage_idx(0, 0)
    stage_idx(1, 1)
    stage_idx(2, 2)
    wait_idx(0)
    pltpu.async_copy(a_hbm.at[srcs[0]], rows[0], sems[0])
    wait_idx(1)
    pltpu.async_copy(a_hbm.at[srcs[1]], rows[1], sems[1])

    def round_body(jo, carry):
        for b in (0, 1, 2):
            j = jo * 3 + b
            b2 = (b + 2) % 3
            pltpu.make_async_copy(a_hbm.at[srcs[b]], rows[b], sems[b]).wait()
            wait_idx(b2)
            pltpu.async_copy(a_hbm.at[srcs[b2]], rows[b2], sems[b2])
            pltpu.sync_copy(rows[b], acc_sh.at[dsts[b]], add=True)
            stage_idx(j + 3, b)
        return carry

    lax.fori_loop(0, (_NCH - 2) // 3, round_body, 0)
    for k in (_NCH - 2, _NCH - 1):
        bk = k % 3
        pltpu.make_async_copy(a_hbm.at[srcs[bk]], rows[bk], sems[bk]).wait()
        pltpu.sync_copy(rows[bk], acc_sh.at[dsts[bk]], add=True)
    wait_idx((_NCH - 3) % 3)
    plsc.subcore_barrier()

    def wback(i, carry):
        o = base + i * _BB
        pltpu.sync_copy(acc_sh.at[pl.ds(o, _BB)], bb_v)
        pltpu.sync_copy(bb_v, accp_hbm.at[c, pl.ds(o, _BB)])
        return carry

    lax.fori_loop(0, _NBB, wback, 0)


_W = 4


@functools.partial(
    pl.kernel,
    out_type=jax.ShapeDtypeStruct((_NC, _NP, _D), jnp.float32),
    mesh=_MESH,
    scratch_types=[
        pltpu.VMEM((_NCH, _CH), jnp.int32),
        pltpu.VMEM((_CH, _D), jnp.float32),
        pltpu.VMEM((_BB, _D), jnp.float32),
        pltpu.SemaphoreType.DMA,
        pltpu.VMEM_SHARED((_NP, _D), jnp.float32),
    ],
)
def _cnt(dst_hbm, zd_hbm, ones_hbm,
         cntp_hbm,
         didx_v, ones_v, bb_v, ssem, cnt_sh):
    c = lax.axis_index("c")
    s = lax.axis_index("s")
    wid = s * _NC + c
    base = s * _RPT
    pltpu.sync_copy(ones_hbm, ones_v)
    pltpu.sync_copy(zd_hbm, bb_v)

    def zinit(i, carry):
        pltpu.sync_copy(bb_v, cnt_sh.at[pl.ds(base + i * _BB, _BB)])
        return carry

    lax.fori_loop(0, _NBB, zinit, 0)
    plsc.subcore_barrier()
    pltpu.sync_copy(dst_hbm.at[wid], didx_v)

    for j in range(_W):
        pltpu.async_copy(ones_v, cnt_sh.at[didx_v.at[j]], add=True, sem=ssem)

    def body(j, carry):
        pltpu.async_copy(ones_v, cnt_sh.at[didx_v.at[j]], add=True, sem=ssem)
        pltpu.make_async_copy(ones_v, cnt_sh.at[didx_v.at[j - _W]], ssem).wait()
        return carry

    lax.fori_loop(_W, _NCH, body, 0)
    for j in range(_W):
        pltpu.make_async_copy(ones_v, cnt_sh.at[didx_v.at[_NCH - _W + j]],
                              ssem).wait()
    plsc.subcore_barrier()

    def wback(i, carry):
        o = base + i * _BB
        pltpu.sync_copy(cnt_sh.at[pl.ds(o, _BB)], bb_v)
        pltpu.sync_copy(bb_v, cntp_hbm.at[c, pl.ds(o, _BB)])
        return carry

    lax.fori_loop(0, _NBB, wback, 0)


_BR = 1000


def _tc1_body(x_ref, s1_ref, cnt_ref, wl1_ref, wr1_ref, b1_ref, wl2_ref,
              h_ref, p2_ref):
    cnt = jnp.maximum(cnt_ref[0][:, 0:1] + cnt_ref[1][:, 0:1], 1.0)
    mean = (s1_ref[0] + s1_ref[1]) / cnt
    h = jnp.dot(mean, wl1_ref[...], preferred_element_type=jnp.float32)
    h = h + jnp.dot(x_ref[...], wr1_ref[...], preferred_element_type=jnp.float32)
    h = jnp.maximum(h + b1_ref[...], 0.0)
    h_ref[...] = h
    p2_ref[...] = jnp.dot(h, wl2_ref[...], preferred_element_type=jnp.float32)


_tc1 = pl.pallas_call(
    _tc1_body,
    grid=(_N // _BR,),
    in_specs=[
        pl.BlockSpec((_BR, _D), lambda i: (i, 0)),
        pl.BlockSpec((_NC, _BR, _D), lambda i: (0, i, 0)),
        pl.BlockSpec((_NC, _BR, _D), lambda i: (0, i, 0)),
        pl.BlockSpec((_D, _HID), lambda i: (0, 0)),
        pl.BlockSpec((_D, _HID), lambda i: (0, 0)),
        pl.BlockSpec((1, _HID), lambda i: (0, 0)),
        pl.BlockSpec((_HID, _EMB), lambda i: (0, 0)),
    ],
    out_specs=[
        pl.BlockSpec((_BR, _HID), lambda i: (i, 0)),
        pl.BlockSpec((_BR, _EMB), lambda i: (i, 0)),
    ],
    out_shape=[
        jax.ShapeDtypeStruct((_N, _HID), jnp.float32),
        jax.ShapeDtypeStruct((_N, _EMB), jnp.float32),
    ],
)


def _tc2_body(h_ref, s2_ref, cnt_ref, wr2_ref, b2_ref, wo_ref, bo_ref,
              out_ref, h2_ref):
    cnt = jnp.maximum(cnt_ref[0][:, 0:1] + cnt_ref[1][:, 0:1], 1.0)
    mean = (s2_ref[0] + s2_ref[1]) / cnt
    h2 = mean + jnp.dot(h_ref[...], wr2_ref[...], preferred_element_type=jnp.float32)
    h2 = h2 + b2_ref[...]
    h2_ref[...] = h2
    out_ref[...] = jnp.dot(h2, wo_ref[...], preferred_element_type=jnp.float32) + bo_ref[...]


_tc2 = pl.pallas_call(
    _tc2_body,
    grid=(_N // _BR,),
    in_specs=[
        pl.BlockSpec((_BR, _HID), lambda i: (i, 0)),
        pl.BlockSpec((_NC, _BR, _D), lambda i: (0, i, 0)),
        pl.BlockSpec((_NC, _BR, _D), lambda i: (0, i, 0)),
        pl.BlockSpec((_HID, _EMB), lambda i: (0, 0)),
        pl.BlockSpec((1, _EMB), lambda i: (0, 0)),
        pl.BlockSpec((_EMB, _OUT), lambda i: (0, 0)),
        pl.BlockSpec((1, _OUT), lambda i: (0, 0)),
    ],
    out_specs=[
        pl.BlockSpec((_BR, _OUT), lambda i: (i, 0)),
        pl.BlockSpec((_BR, _EMB), lambda i: (i, 0)),
    ],
    out_shape=[
        jax.ShapeDtypeStruct((_N, _OUT), jnp.float32),
        jax.ShapeDtypeStruct((_N, _EMB), jnp.float32),
    ],
)


def kernel(x, edge_index, W_l1, W_r1, b1, W_l2, W_r2, b2, W_o, b_o):
    src = edge_index[0]
    dst = edge_index[1]
    dst3 = dst.reshape(_NW, _NCH, _CH)
    zd = jnp.zeros((_BB, _D), jnp.float32)
    ones = jnp.ones((_CH, _D), jnp.float32)
    cntp = _cnt(dst3, zd, ones)
    s1p = _seg(x, src, dst, zd)
    h, p2 = _tc1(x, s1p, cntp, W_l1, W_r1, b1.reshape(1, _HID), W_l2)
    s2p = _seg(p2, src, dst, zd)
    out, h2 = _tc2(h, s2p, cntp, W_r2, b2.reshape(1, _EMB), W_o,
                   b_o.reshape(1, _OUT))
    return (out, h2)
